# baseline (speedup 1.0000x reference)
import numpy as np
import jax
import jax.numpy as jnp
from jax.experimental import pallas as pl

SLOWFAST_ALPHA = 4


def kernel(frames):
    C, T, H, W = frames.shape
    n = T // SLOWFAST_ALPHA
    idx = np.trunc(np.linspace(0.0, T - 1, n)).astype(np.int32)
    HW = H * W
    x = frames.reshape(C, T, 1, HW)

    def slow_index(t):
        s = jnp.int32(0)
        for j in idx:
            s = s + (t > j).astype(jnp.int32)
        return (0, s, 0, 0)

    sel_set = set(int(v) for v in idx)

    def body(x_ref, slow_ref, fast_ref):
        t = pl.program_id(0)
        v = x_ref[...]
        fast_ref[...] = v

        sel = (t == int(idx[0]))
        for j in sorted(sel_set - {int(idx[0])}):
            sel = sel | (t == j)

        @pl.when(sel)
        def _():
            slow_ref[...] = v

    slow, fast = pl.pallas_call(
        body,
        grid=(T,),
        in_specs=[pl.BlockSpec((C, 1, 1, HW), lambda t: (0, t, 0, 0))],
        out_specs=[
            pl.BlockSpec((C, 1, 1, HW), slow_index),
            pl.BlockSpec((C, 1, 1, HW), lambda t: (0, t, 0, 0)),
        ],
        out_shape=[
            jax.ShapeDtypeStruct((C, n, 1, HW), frames.dtype),
            jax.ShapeDtypeStruct((C, T, 1, HW), frames.dtype),
        ],
    )(x)
    return (slow.reshape(C, n, H, W), fast.reshape(C, T, H, W))


# speedup vs baseline: 4.6870x; 4.6870x over previous
import numpy as np
import jax
import jax.numpy as jnp
from jax.experimental import pallas as pl

SLOWFAST_ALPHA = 4


def kernel(frames):
    C, T, H, W = frames.shape
    n = T // SLOWFAST_ALPHA
    idx = np.trunc(np.linspace(0.0, T - 1, n)).astype(np.int32)
    HW = H * W
    x = frames.reshape(C, T, HW // 128, 128)

    def slow_index(t):
        s = jnp.int32(0)
        for j in idx:
            s = s + (t > j).astype(jnp.int32)
        return (0, s, 0, 0)

    sel_set = set(int(v) for v in idx)

    def body(x_ref, slow_ref, fast_ref):
        t = pl.program_id(0)
        v = x_ref[...]
        fast_ref[...] = v

        sel = (t == int(idx[0]))
        for j in sorted(sel_set - {int(idx[0])}):
            sel = sel | (t == j)

        @pl.when(sel)
        def _():
            slow_ref[...] = v

    slow, fast = pl.pallas_call(
        body,
        grid=(T,),
        in_specs=[pl.BlockSpec((C, 1, HW // 128, 128), lambda t: (0, t, 0, 0))],
        out_specs=[
            pl.BlockSpec((C, 1, HW // 128, 128), slow_index),
            pl.BlockSpec((C, 1, HW // 128, 128), lambda t: (0, t, 0, 0)),
        ],
        out_shape=[
            jax.ShapeDtypeStruct((C, n, HW // 128, 128), frames.dtype),
            jax.ShapeDtypeStruct((C, T, HW // 128, 128), frames.dtype),
        ],
    )(x)
    return (slow.reshape(C, n, H, W), fast.reshape(C, T, H, W))
